# baseline (speedup 1.0000x reference)
import functools

import jax
import jax.numpy as jnp
from jax import lax
from jax.experimental import pallas as pl
from jax.experimental.pallas import tpu as pltpu
from jax.experimental.pallas import tpu_sc as plsc

B = 1024
L = 200
D = 768
LANES = 16
DV = D // LANES
CHUNK = 40
CHUNKS_PER_ROW = L // CHUNK


def _sc_body(nc, rows_per_w, wt_hbm, tok_hbm, b_hbm, out_hbm, mask_hbm,
             toks_v, buf_v, acc_v, bias_v, mask_v, sem):
    wid = lax.axis_index("s") * nc + lax.axis_index("c")
    base_row = wid * rows_per_w
    tok_per_w = rows_per_w * L
    nchunk = tok_per_w // CHUNK

    pltpu.sync_copy(tok_hbm.at[pl.ds(base_row * L, tok_per_w)], toks_v)
    pltpu.sync_copy(b_hbm, bias_v)

    def init_row(r, carry):
        for c in range(DV):
            acc_v[r, pl.ds(c * LANES, LANES)] = bias_v[pl.ds(c * LANES, LANES)]
        return carry

    lax.fori_loop(0, rows_per_w, init_row, 0)

    def chunk_body(g, carry):
        r = g // CHUNKS_PER_ROW
        off = g * CHUNK
        idx = toks_v.at[pl.ds(off, CHUNK)]
        pltpu.async_copy(wt_hbm.at[idx], buf_v, sem).wait()

        def jbody(j, c2):
            for c in range(DV):
                co = c * LANES
                plsc.addupdate(acc_v.at[r, pl.ds(co, LANES)],
                               buf_v[j, pl.ds(co, LANES)])
            return c2

        lax.fori_loop(0, CHUNK, jbody, 0)
        return carry

    lax.fori_loop(0, nchunk, chunk_body, 0)

    def out_row(r, carry):
        s = acc_v[r, pl.ds(0, LANES)]
        for c in range(1, DV):
            s = s + acc_v[r, pl.ds(c * LANES, LANES)]
        mask_v[r, :] = s
        return carry

    lax.fori_loop(0, rows_per_w, out_row, 0)

    pltpu.sync_copy(acc_v, out_hbm.at[pl.ds(base_row, rows_per_w)])
    pltpu.sync_copy(mask_v, mask_hbm.at[pl.ds(base_row, rows_per_w)])


def kernel(token_ids, W, b):
    info = plsc.get_sparse_core_info()
    nc, ns = info.num_cores, info.num_subcores
    nw = nc * ns
    rows_per_w = B // nw

    wt = W.T
    toks = token_ids.reshape(-1).astype(jnp.int32)

    mesh = plsc.VectorSubcoreMesh(core_axis_name="c", subcore_axis_name="s")
    sc = pl.kernel(
        functools.partial(_sc_body, nc, rows_per_w),
        out_type=(
            jax.ShapeDtypeStruct((B, D), jnp.float32),
            jax.ShapeDtypeStruct((B, LANES), jnp.float32),
        ),
        mesh=mesh,
        scratch_types=[
            pltpu.VMEM((rows_per_w * L,), jnp.int32),
            pltpu.VMEM((CHUNK, D), jnp.float32),
            pltpu.VMEM((rows_per_w, D), jnp.float32),
            pltpu.VMEM((D,), jnp.float32),
            pltpu.VMEM((rows_per_w, LANES), jnp.float32),
            pltpu.SemaphoreType.DMA,
        ],
    )
    out2d, sums16 = sc(wt, toks, b)
    padding_mask = jnp.sum(sums16, axis=1, keepdims=True) == 0.0
    return (out2d[:, None, :], padding_mask)


# speedup vs baseline: 1.2644x; 1.2644x over previous
import functools

import jax
import jax.numpy as jnp
from jax import lax
from jax.experimental import pallas as pl
from jax.experimental.pallas import tpu as pltpu
from jax.experimental.pallas import tpu_sc as plsc

B = 1024
L = 200
D = 768
LANES = 16
DV = D // LANES
CHUNK = 40
CPR = L // CHUNK
NSLOTS = 2


def _sc_body(nc, ns, wt_hbm, tok_hbm, b_hbm, out_hbm, sums_hbm,
             toks_v, buf0, buf1, acc_v, bias_v, sums_v, sg0, sg1):
    nw = nc * ns
    rows_w = B // nw
    tok_w = rows_w * L
    nchunk = tok_w // CHUNK

    cid = lax.axis_index("c")
    sid = lax.axis_index("s")
    wid = cid * ns + sid
    grow = wid * rows_w

    bufs = (buf0, buf1)
    sgs = (sg0, sg1)

    pltpu.sync_copy(tok_hbm.at[pl.ds(wid * tok_w, tok_w)], toks_v)
    pltpu.sync_copy(b_hbm, bias_v)

    def fill_row(r, carry):
        for c in range(DV):
            acc_v[r, pl.ds(c * LANES, LANES)] = bias_v[pl.ds(c * LANES, LANES)]
        return carry

    lax.fori_loop(0, rows_w, fill_row, 0)

    def start_gather(g, u):
        pltpu.async_copy(wt_hbm.at[toks_v.at[pl.ds(g * CHUNK, CHUNK)]],
                         bufs[u], sgs[u])

    for u in range(NSLOTS):
        start_gather(u, u)

    def go_body(go, carry):
        for u in range(NSLOTS):
            g = go * NSLOTS + u
            r = g // CPR
            pltpu.make_async_copy(
                wt_hbm.at[toks_v.at[pl.ds(g * CHUNK, CHUNK)]],
                bufs[u], sgs[u]).wait()

            def jbody(j, c2):
                for c in range(DV):
                    co = c * LANES
                    plsc.addupdate(acc_v.at[r, pl.ds(co, LANES)],
                                   bufs[u][j, pl.ds(co, LANES)])
                return c2

            lax.fori_loop(0, CHUNK, jbody, 0)

            @pl.when(g + NSLOTS < nchunk)
            def _():
                start_gather(g + NSLOTS, u)
        return carry

    lax.fori_loop(0, nchunk // NSLOTS, go_body, 0)

    def out_row(r, carry):
        s = acc_v[r, pl.ds(0, LANES)]
        for c in range(1, DV):
            s = s + acc_v[r, pl.ds(c * LANES, LANES)]
        sums_v[pl.ds(r * LANES, LANES)] = s
        return carry

    lax.fori_loop(0, rows_w, out_row, 0)

    pltpu.sync_copy(acc_v, out_hbm.at[pl.ds(grow, rows_w)])
    pltpu.sync_copy(sums_v, sums_hbm.at[pl.ds(wid * rows_w * LANES,
                                              rows_w * LANES)])


def kernel(token_ids, W, b):
    info = plsc.get_sparse_core_info()
    nc, ns = info.num_cores, info.num_subcores
    nw = nc * ns
    rows_w = B // nw
    assert (rows_w * L) % (CHUNK * NSLOTS) == 0

    wt = W.T
    toks = token_ids.reshape(-1).astype(jnp.int32)

    mesh = plsc.VectorSubcoreMesh(core_axis_name="c", subcore_axis_name="s")
    sc = pl.kernel(
        functools.partial(_sc_body, nc, ns),
        out_type=(
            jax.ShapeDtypeStruct((B, D), jnp.float32),
            jax.ShapeDtypeStruct((B * LANES,), jnp.float32),
        ),
        mesh=mesh,
        scratch_types=[
            pltpu.VMEM((rows_w * L,), jnp.int32),
            pltpu.VMEM((CHUNK, D), jnp.float32),
            pltpu.VMEM((CHUNK, D), jnp.float32),
            pltpu.VMEM((rows_w, D), jnp.float32),
            pltpu.VMEM((D,), jnp.float32),
            pltpu.VMEM((rows_w * LANES,), jnp.float32),
            pltpu.SemaphoreType.DMA,
            pltpu.SemaphoreType.DMA,
        ],
    )
    out2d, sums = sc(wt, toks, b)
    padding_mask = jnp.sum(sums.reshape(B, LANES), axis=1, keepdims=True) == 0.0
    return (out2d[:, None, :], padding_mask)


# speedup vs baseline: 4.2318x; 3.3469x over previous
import functools

import jax
import jax.numpy as jnp
from jax import lax
from jax.experimental import pallas as pl
from jax.experimental.pallas import tpu as pltpu
from jax.experimental.pallas import tpu_sc as plsc

B = 1024
L = 200
D = 768
LANES = 16
DV = D // LANES
CHUNK = 40
CPR = L // CHUNK
NSLOTS = 2


def _sc_body(nc, ns, wt_hbm, tok_hbm, b_hbm, out_hbm, sums_hbm,
             toks_v, buf0, buf1, acc_v, bias_v, sums_v, sg0, sg1):
    nw = nc * ns
    rows_w = B // nw
    tok_w = rows_w * L
    nchunk = tok_w // CHUNK

    cid = lax.axis_index("c")
    sid = lax.axis_index("s")
    wid = cid * ns + sid
    grow = wid * rows_w

    bufs = (buf0, buf1)
    sgs = (sg0, sg1)

    pltpu.sync_copy(tok_hbm.at[pl.ds(wid * tok_w, tok_w)], toks_v)
    pltpu.sync_copy(b_hbm, bias_v)

    def fill_row(r, carry):
        for c in range(DV):
            acc_v[r, pl.ds(c * LANES, LANES)] = bias_v[pl.ds(c * LANES, LANES)]
        return carry

    lax.fori_loop(0, rows_w, fill_row, 0)

    def start_gather(g, u):
        pltpu.async_copy(wt_hbm.at[toks_v.at[pl.ds(g * CHUNK, CHUNK)]],
                         bufs[u], sgs[u])

    for u in range(NSLOTS):
        start_gather(u, u)

    def go_body(go, carry):
        for u in range(NSLOTS):
            g = go * NSLOTS + u
            r = g // CPR
            pltpu.make_async_copy(
                wt_hbm.at[toks_v.at[pl.ds(g * CHUNK, CHUNK)]],
                bufs[u], sgs[u]).wait()

            def cbody(c, c2):
                co = pl.multiple_of(c * LANES, LANES)
                vals = [bufs[u][j, pl.ds(co, LANES)] for j in range(CHUNK)]
                while len(vals) > 1:
                    nxt = [vals[i] + vals[i + 1]
                           for i in range(0, len(vals) - 1, 2)]
                    if len(vals) % 2:
                        nxt.append(vals[-1])
                    vals = nxt
                plsc.addupdate(acc_v.at[r, pl.ds(co, LANES)], vals[0])
                return c2

            lax.fori_loop(0, DV, cbody, 0)

            @pl.when(g + NSLOTS < nchunk)
            def _():
                start_gather(g + NSLOTS, u)
        return carry

    lax.fori_loop(0, nchunk // NSLOTS, go_body, 0)

    def out_row(r, carry):
        s = acc_v[r, pl.ds(0, LANES)]
        for c in range(1, DV):
            s = s + acc_v[r, pl.ds(c * LANES, LANES)]
        sums_v[pl.ds(r * LANES, LANES)] = s
        return carry

    lax.fori_loop(0, rows_w, out_row, 0)

    pltpu.sync_copy(acc_v, out_hbm.at[pl.ds(grow, rows_w)])
    pltpu.sync_copy(sums_v, sums_hbm.at[pl.ds(wid * rows_w * LANES,
                                              rows_w * LANES)])


def kernel(token_ids, W, b):
    info = plsc.get_sparse_core_info()
    nc, ns = info.num_cores, info.num_subcores
    nw = nc * ns
    rows_w = B // nw
    assert (rows_w * L) % (CHUNK * NSLOTS) == 0

    wt = W.T
    toks = token_ids.reshape(-1).astype(jnp.int32)

    mesh = plsc.VectorSubcoreMesh(core_axis_name="c", subcore_axis_name="s")
    sc = pl.kernel(
        functools.partial(_sc_body, nc, ns),
        out_type=(
            jax.ShapeDtypeStruct((B, D), jnp.float32),
            jax.ShapeDtypeStruct((B * LANES,), jnp.float32),
        ),
        mesh=mesh,
        scratch_types=[
            pltpu.VMEM((rows_w * L,), jnp.int32),
            pltpu.VMEM((CHUNK, D), jnp.float32),
            pltpu.VMEM((CHUNK, D), jnp.float32),
            pltpu.VMEM((rows_w, D), jnp.float32),
            pltpu.VMEM((D,), jnp.float32),
            pltpu.VMEM((rows_w * LANES,), jnp.float32),
            pltpu.SemaphoreType.DMA,
            pltpu.SemaphoreType.DMA,
        ],
    )
    out2d, sums = sc(wt, toks, b)
    padding_mask = jnp.sum(sums.reshape(B, LANES), axis=1, keepdims=True) == 0.0
    return (out2d[:, None, :], padding_mask)
